# baseline (speedup 1.0000x reference)
import jax
import jax.numpy as jnp
from jax.experimental import pallas as pl


def _routing_kernel(x_ref, w1t_ref, b1_ref, w2t_ref, b2_ref,
                    probs_ref, oh0_ref, oh1_ref, ct1_ref, ct2_ref,
                    a_ref, psum_ref, aux_ref):
    i = pl.program_id(0)
    nblk = pl.num_programs(0)
    x = x_ref[...]
    h = jnp.dot(x, w1t_ref[...], preferred_element_type=jnp.float32)
    h = jnp.maximum(h + b1_ref[...], 0.0)
    logits = jnp.dot(h, w2t_ref[...], preferred_element_type=jnp.float32)
    logits = logits + b2_ref[...]

    m = jnp.max(logits, axis=-1, keepdims=True)
    ex = jnp.exp(logits - m)
    probs = ex / jnp.sum(ex, axis=-1, keepdims=True)
    probs_ref[...] = probs

    S, E = probs.shape
    iota = jax.lax.broadcasted_iota(jnp.int32, (S, E), 1)
    e0 = jnp.argmax(probs, axis=-1)
    oh0 = iota == e0[:, None]
    p0 = jnp.max(probs, axis=-1, keepdims=True)
    masked = jnp.where(oh0, -1.0, probs)
    e1 = jnp.argmax(masked, axis=-1)
    oh1 = iota == e1[:, None]
    p1 = jnp.max(masked, axis=-1, keepdims=True)
    tot = p0 + p1

    oh0f = oh0.astype(jnp.float32)
    oh1f = oh1.astype(jnp.float32)
    oh0_ref[...] = oh0f
    oh1_ref[...] = oh1f
    ct1_ref[...] = oh0f * (p0 / tot)
    ct2_ref[...] = oh1f * (p1 / tot)

    blk_a = jnp.max(oh0f, axis=0, keepdims=True)
    blk_s = jnp.sum(probs, axis=0, keepdims=True)

    @pl.when(i == 0)
    def _init():
        a_ref[...] = blk_a
        psum_ref[...] = blk_s

    @pl.when(i > 0)
    def _acc():
        a_ref[...] = jnp.maximum(a_ref[...], blk_a)
        psum_ref[...] = psum_ref[...] + blk_s

    @pl.when(i == nblk - 1)
    def _fin():
        mean_probs = psum_ref[...] / (S * nblk)
        aux_ref[...] = jnp.sum(mean_probs * jnp.log(mean_probs * E + 1e-9),
                               axis=-1, keepdims=True)


def _expand_kernel(oh0_ref, oh1_ref, ct1_ref, ct2_ref, a_ref,
                   disp_ref, comb_ref):
    t, e, cap = disp_ref.shape
    L = 128
    i = pl.program_id(0)
    rows = pl.ds(i * t, t)
    a = a_ref[...]
    oh0 = oh0_ref[rows, :]
    oh1 = oh1_ref[rows, :]
    ct1 = ct1_ref[rows, :]
    ct2 = ct2_ref[rows, :]
    sec1 = oh1 * a
    sec0 = oh1 - sec1
    d0 = (oh0 + sec0)[:, :, None]
    d1 = sec1[:, :, None]
    c1b = ct2 * a
    c0 = (ct1 + (ct2 - c1b))[:, :, None]
    c1 = c1b[:, :, None]
    ci = jax.lax.broadcasted_iota(jnp.int32, (t, e, L), 2)
    is0 = ci == 0
    is1 = ci == 1
    disp_ref[:, :, :L] = jnp.where(is0, d0, jnp.where(is1, d1, 0.0))
    comb_ref[:, :, :L] = jnp.where(is0, c0, jnp.where(is1, c1, 0.0))
    tail = jnp.zeros((t, e, cap - L), jnp.float32)
    disp_ref[:, :, L:] = tail
    comb_ref[:, :, L:] = tail


def kernel(hidden_states, W1, b1, W2, b2):
    B, S, H = hidden_states.shape
    E = W2.shape[0]
    k = 2
    capacity = int(B * S * 1.5 * k / E)
    N = B * S

    x = hidden_states.reshape(N, H)
    w1t = W1.T
    w2t = W2.T
    b1r = b1.reshape(1, H)
    b2r = b2.reshape(1, E)

    T1 = 256
    nblk1 = N // T1
    row_spec = pl.BlockSpec((T1, E), lambda i: (i, 0))
    vec_spec = pl.BlockSpec((1, E), lambda i: (0, 0))
    probs, oh0, oh1, ct1, ct2, a, _psum, aux = pl.pallas_call(
        _routing_kernel,
        grid=(nblk1,),
        in_specs=[
            pl.BlockSpec((T1, H), lambda i: (i, 0)),
            pl.BlockSpec((H, H), lambda i: (0, 0)),
            pl.BlockSpec((1, H), lambda i: (0, 0)),
            pl.BlockSpec((H, E), lambda i: (0, 0)),
            pl.BlockSpec((1, E), lambda i: (0, 0)),
        ],
        out_specs=[row_spec, row_spec, row_spec, row_spec, row_spec,
                   vec_spec, vec_spec, pl.BlockSpec((1, 1), lambda i: (0, 0))],
        out_shape=[
            jax.ShapeDtypeStruct((N, E), jnp.float32),
            jax.ShapeDtypeStruct((N, E), jnp.float32),
            jax.ShapeDtypeStruct((N, E), jnp.float32),
            jax.ShapeDtypeStruct((N, E), jnp.float32),
            jax.ShapeDtypeStruct((N, E), jnp.float32),
            jax.ShapeDtypeStruct((1, E), jnp.float32),
            jax.ShapeDtypeStruct((1, E), jnp.float32),
            jax.ShapeDtypeStruct((1, 1), jnp.float32),
        ],
    )(x, w1t, b1r, w2t, b2r)

    T = 128
    nblk = N // T
    plane_spec = pl.BlockSpec((N, E), lambda i: (0, 0))
    out_spec = pl.BlockSpec((T, E, capacity), lambda i: (i, 0, 0))
    dispatch, combine = pl.pallas_call(
        _expand_kernel,
        grid=(nblk,),
        in_specs=[plane_spec, plane_spec, plane_spec, plane_spec,
                  pl.BlockSpec((1, E), lambda i: (0, 0))],
        out_specs=[out_spec, out_spec],
        out_shape=[
            jax.ShapeDtypeStruct((N, E, capacity), jnp.float32),
            jax.ShapeDtypeStruct((N, E, capacity), jnp.float32),
        ],
    )(oh0, oh1, ct1, ct2, a)

    return (dispatch.reshape(B, S, E, capacity),
            combine.reshape(B, S, E, capacity),
            probs.reshape(B, S, E),
            aux[0, 0])


# speedup vs baseline: 1.0657x; 1.0657x over previous
import functools

import jax
import jax.numpy as jnp
from jax.experimental import pallas as pl
from jax.experimental.pallas import tpu as pltpu


def _fused_kernel(x_ref, w1t_ref, b1_ref, w2t_ref, b2_ref,
                  probs_ref, aux_ref, disp_ref, comb_ref,
                  oh0_s, oh1_s, ct1_s, ct2_s, av_s, ps_s,
                  *, nr, t1):
    i = pl.program_id(0)

    @pl.when(i < nr)
    def _route():
        x = x_ref[...]
        h = jnp.dot(x, w1t_ref[...], preferred_element_type=jnp.float32)
        h = jnp.maximum(h + b1_ref[...], 0.0)
        logits = jnp.dot(h, w2t_ref[...], preferred_element_type=jnp.float32)
        logits = logits + b2_ref[...]

        m = jnp.max(logits, axis=-1, keepdims=True)
        ex = jnp.exp(logits - m)
        probs = ex / jnp.sum(ex, axis=-1, keepdims=True)
        probs_ref[...] = probs

        S, E = probs.shape
        iota = jax.lax.broadcasted_iota(jnp.int32, (S, E), 1)
        e0 = jnp.argmax(probs, axis=-1)
        oh0 = iota == e0[:, None]
        p0 = jnp.max(probs, axis=-1, keepdims=True)
        masked = jnp.where(oh0, -1.0, probs)
        e1 = jnp.argmax(masked, axis=-1)
        oh1 = iota == e1[:, None]
        p1 = jnp.max(masked, axis=-1, keepdims=True)
        tot = p0 + p1

        oh0f = oh0.astype(jnp.float32)
        oh1f = oh1.astype(jnp.float32)
        rows = pl.ds(i * t1, t1)
        oh0_s[rows, :] = oh0f
        oh1_s[rows, :] = oh1f
        ct1_s[rows, :] = oh0f * (p0 / tot)
        ct2_s[rows, :] = oh1f * (p1 / tot)

        blk_a = jnp.max(oh0f, axis=0, keepdims=True)
        blk_s = jnp.sum(probs, axis=0, keepdims=True)

        @pl.when(i == 0)
        def _init():
            av_s[...] = blk_a
            ps_s[...] = blk_s

        @pl.when(i > 0)
        def _acc():
            av_s[...] = jnp.maximum(av_s[...], blk_a)
            ps_s[...] = ps_s[...] + blk_s

        @pl.when(i == nr - 1)
        def _fin():
            mean_probs = ps_s[...] / (t1 * nr)
            aux_ref[...] = jnp.sum(
                mean_probs * jnp.log(mean_probs * E + 1e-9),
                axis=-1, keepdims=True)

    @pl.when(i >= nr)
    def _write():
        t, e, cap = disp_ref.shape
        L = 128
        rows = pl.ds((i - nr) * t, t)
        a = av_s[...]
        oh0 = oh0_s[rows, :]
        oh1 = oh1_s[rows, :]
        ct1 = ct1_s[rows, :]
        ct2 = ct2_s[rows, :]
        sec1 = oh1 * a
        d0 = (oh0 + (oh1 - sec1))[:, :, None]
        d1 = sec1[:, :, None]
        c1b = ct2 * a
        c0 = (ct1 + (ct2 - c1b))[:, :, None]
        c1 = c1b[:, :, None]
        ci = jax.lax.broadcasted_iota(jnp.int32, (t, e, L), 2)
        is0 = ci == 0
        is1 = ci == 1
        disp_ref[:, :, :L] = jnp.where(is0, d0, jnp.where(is1, d1, 0.0))
        comb_ref[:, :, :L] = jnp.where(is0, c0, jnp.where(is1, c1, 0.0))
        tail = jnp.zeros((t, e, cap - L), jnp.float32)
        disp_ref[:, :, L:] = tail
        comb_ref[:, :, L:] = tail


def kernel(hidden_states, W1, b1, W2, b2):
    B, S, H = hidden_states.shape
    E = W2.shape[0]
    k = 2
    capacity = int(B * S * 1.5 * k / E)
    N = B * S

    x = hidden_states.reshape(N, H)
    w1t = W1.T
    w2t = W2.T
    b1r = b1.reshape(1, H)
    b2r = b2.reshape(1, E)

    T1 = 256
    nr = N // T1
    TW = 128
    nw = N // TW
    last_r = nr - 1

    probs, aux, dispatch, combine = pl.pallas_call(
        functools.partial(_fused_kernel, nr=nr, t1=T1),
        grid=(nr + nw,),
        in_specs=[
            pl.BlockSpec((T1, H), lambda i: (jnp.minimum(i, last_r), 0)),
            pl.BlockSpec((H, H), lambda i: (0, 0)),
            pl.BlockSpec((1, H), lambda i: (0, 0)),
            pl.BlockSpec((H, E), lambda i: (0, 0)),
            pl.BlockSpec((1, E), lambda i: (0, 0)),
        ],
        out_specs=[
            pl.BlockSpec((T1, E), lambda i: (jnp.minimum(i, last_r), 0)),
            pl.BlockSpec((1, 1), lambda i: (0, 0)),
            pl.BlockSpec((TW, E, capacity),
                         lambda i: (jnp.maximum(i - nr, 0), 0, 0)),
            pl.BlockSpec((TW, E, capacity),
                         lambda i: (jnp.maximum(i - nr, 0), 0, 0)),
        ],
        out_shape=[
            jax.ShapeDtypeStruct((N, E), jnp.float32),
            jax.ShapeDtypeStruct((1, 1), jnp.float32),
            jax.ShapeDtypeStruct((N, E, capacity), jnp.float32),
            jax.ShapeDtypeStruct((N, E, capacity), jnp.float32),
        ],
        scratch_shapes=[
            pltpu.VMEM((N, E), jnp.float32),
            pltpu.VMEM((N, E), jnp.float32),
            pltpu.VMEM((N, E), jnp.float32),
            pltpu.VMEM((N, E), jnp.float32),
            pltpu.VMEM((1, E), jnp.float32),
            pltpu.VMEM((1, E), jnp.float32),
        ],
    )(x, w1t, b1r, w2t, b2r)

    return (dispatch.reshape(B, S, E, capacity),
            combine.reshape(B, S, E, capacity),
            probs.reshape(B, S, E),
            aux[0, 0])
